# baseline (speedup 1.0000x reference)
import functools

import jax
import jax.numpy as jnp
from jax import lax
from jax.experimental import pallas as pl
from jax.experimental.pallas import tpu as pltpu
from jax.experimental.pallas import tpu_sc as plsc

N = 10000
DEG = 32
D = 128
NC = 2
NS = 16
NW = NC * NS
L = 16

CB = 8
HALF = CB // 2
IDX = CB * DEG
NCHUNK = N // CB
CPW = NCHUNK // NW
REM = NCHUNK - CPW * NW
MAXC = CPW + 1
NB = 3

_mesh = plsc.VectorSubcoreMesh(core_axis_name="c", subcore_axis_name="s")


@functools.partial(
    pl.kernel,
    mesh=_mesh,
    out_type=jax.ShapeDtypeStruct((N, D), jnp.float32),
    scratch_types=[
        pltpu.VMEM((MAXC * IDX,), jnp.int32),
        pltpu.VMEM((NB, 2, 128, D), jnp.float32),
        pltpu.VMEM((NB, CB, D), jnp.float32),
        pltpu.SemaphoreType.DMA,
        pltpu.SemaphoreType.DMA,
        pltpu.SemaphoreType.DMA,
        pltpu.SemaphoreType.DMA,
        pltpu.SemaphoreType.DMA,
        pltpu.SemaphoreType.DMA,
    ],
)
def _aggregate(idx_hbm, table_hbm, out_hbm, idx_v, rows_v, acc_v,
               gsem0, gsem1, gsem2, osem0, osem1, osem2):
    c32 = jnp.int32
    wid = lax.axis_index("s") * c32(NC) + lax.axis_index("c")
    start = wid * c32(CPW) + jnp.minimum(wid, c32(REM))
    cnt = c32(CPW) + jnp.where(wid < c32(REM), c32(1), c32(0))
    base = jnp.minimum(start, c32(NCHUNK - MAXC))
    off = start - base
    pltpu.sync_copy(idx_hbm.at[pl.ds(base * c32(IDX), MAXC * IDX)], idx_v)

    gsems = (gsem0, gsem1, gsem2)
    osems = (osem0, osem1, osem2)

    def gather_half(i, b, h):
        return pltpu.make_async_copy(
            table_hbm.at[idx_v.at[pl.ds((off + i) * c32(IDX) + c32(h * 128),
                                        128)]],
            rows_v.at[c32(b), c32(h)], gsems[b])

    class _Gather:
        def __init__(self, i, b):
            self.descs = [gather_half(i, b, h) for h in range(2)]

        def start(self):
            for d in self.descs:
                d.start()

        def wait(self):
            for d in self.descs:
                d.wait()

    gather = _Gather

    def outcopy(i, b):
        return pltpu.make_async_copy(
            acc_v.at[c32(b)], out_hbm.at[pl.ds((start + i) * c32(CB), CB)],
            osems[b])

    for b in range(NB):
        gather(c32(b), b).start()

    def ring(j, _):
        for b in range(NB):
            i = j * c32(NB) + c32(b)

            @pl.when(i < cnt)
            def _():
                gather(i, b).wait()

                @pl.when(i >= c32(NB))
                def _():
                    outcopy(i - c32(NB), b).wait()

                for h in range(2):
                    def node_body(n, carry, h=h):
                        r0 = n * c32(DEG)
                        accs = [rows_v[b, h, r0, pl.ds(v * L, L)]
                                for v in range(D // L)]
                        for g in range(1, DEG):
                            for v in range(D // L):
                                accs[v] = accs[v] + rows_v[
                                    b, h, r0 + c32(g), pl.ds(v * L, L)]
                        for v in range(D // L):
                            acc_v[b, c32(h * HALF) + n, pl.ds(v * L, L)] = (
                                accs[v] * (1.0 / DEG))
                        return carry

                    lax.fori_loop(c32(0), c32(HALF), node_body, c32(0))
                outcopy(i, b).start()

                @pl.when(i + c32(NB) < cnt)
                def _():
                    gather(i + c32(NB), b).start()
        return c32(0)

    lax.fori_loop(c32(0), c32((MAXC + NB - 1) // NB), ring, c32(0))
    for b in range(NB):
        outcopy(c32(0), b).wait()


def kernel(nodes, to_neighs, u2e_weight):
    del nodes
    idx = to_neighs.reshape(-1).astype(jnp.int32)
    table = u2e_weight.astype(jnp.float32)
    return _aggregate(idx, table)


# speedup vs baseline: 1.0681x; 1.0681x over previous
import functools

import jax
import jax.numpy as jnp
from jax import lax
from jax.experimental import pallas as pl
from jax.experimental.pallas import tpu as pltpu
from jax.experimental.pallas import tpu_sc as plsc

N = 10000
DEG = 32
D = 128
NC = 2
NS = 16
NW = NC * NS
L = 16

CB = 4
IDX = CB * DEG
NCHUNK = N // CB
CPW = NCHUNK // NW
REM = NCHUNK - CPW * NW
MAXC = CPW + 1
NB = 4

_mesh = plsc.VectorSubcoreMesh(core_axis_name="c", subcore_axis_name="s")


@functools.partial(
    pl.kernel,
    mesh=_mesh,
    out_type=jax.ShapeDtypeStruct((N, D), jnp.float32),
    scratch_types=[
        pltpu.VMEM((MAXC * IDX,), jnp.int32),
        pltpu.VMEM((NB, IDX, D), jnp.float32),
        pltpu.VMEM((NB, CB, D), jnp.float32),
        pltpu.SemaphoreType.DMA,
        pltpu.SemaphoreType.DMA,
        pltpu.SemaphoreType.DMA,
        pltpu.SemaphoreType.DMA,
        pltpu.SemaphoreType.DMA,
        pltpu.SemaphoreType.DMA,
        pltpu.SemaphoreType.DMA,
        pltpu.SemaphoreType.DMA,
    ],
)
def _aggregate(idx_hbm, table_hbm, out_hbm, idx_v, rows_v, acc_v,
               gsem0, gsem1, gsem2, gsem3, osem0, osem1, osem2, osem3):
    c32 = jnp.int32
    wid = lax.axis_index("s") * c32(NC) + lax.axis_index("c")
    start = wid * c32(CPW) + jnp.minimum(wid, c32(REM))
    cnt = c32(CPW) + jnp.where(wid < c32(REM), c32(1), c32(0))
    base = jnp.minimum(start, c32(NCHUNK - MAXC))
    off = start - base
    pltpu.sync_copy(idx_hbm.at[pl.ds(base * c32(IDX), MAXC * IDX)], idx_v)

    gsems = (gsem0, gsem1, gsem2, gsem3)
    osems = (osem0, osem1, osem2, osem3)

    def gather(i, b):
        return pltpu.make_async_copy(
            table_hbm.at[idx_v.at[pl.ds((off + i) * c32(IDX), IDX)]],
            rows_v.at[c32(b)], gsems[b])

    def outcopy(i, b):
        return pltpu.make_async_copy(
            acc_v.at[c32(b)], out_hbm.at[pl.ds((start + i) * c32(CB), CB)],
            osems[b])

    for b in range(NB):
        gather(c32(b), b).start()

    def ring(j, _):
        for b in range(NB):
            i = j * c32(NB) + c32(b)

            @pl.when(i < cnt)
            def _():
                gather(i, b).wait()

                @pl.when(i >= c32(NB))
                def _():
                    outcopy(i - c32(NB), b).wait()

                def node_body(n, carry):
                    r0 = n * c32(DEG)
                    accs = [rows_v[b, r0, pl.ds(v * L, L)]
                            for v in range(D // L)]
                    for g in range(1, DEG):
                        for v in range(D // L):
                            accs[v] = accs[v] + rows_v[b, r0 + c32(g),
                                                       pl.ds(v * L, L)]
                    for v in range(D // L):
                        acc_v[b, n, pl.ds(v * L, L)] = accs[v] * (1.0 / DEG)
                    return carry

                lax.fori_loop(c32(0), c32(CB), node_body, c32(0))
                outcopy(i, b).start()

                @pl.when(i + c32(NB) < cnt)
                def _():
                    gather(i + c32(NB), b).start()
        return c32(0)

    lax.fori_loop(c32(0), c32((MAXC + NB - 1) // NB), ring, c32(0))
    for b in range(NB):
        outcopy(c32(0), b).wait()


def kernel(nodes, to_neighs, u2e_weight):
    del nodes
    idx = to_neighs.reshape(-1).astype(jnp.int32)
    table = u2e_weight.astype(jnp.float32)
    return _aggregate(idx, table)


# speedup vs baseline: 1.3292x; 1.2445x over previous
import functools

import jax
import jax.numpy as jnp
from jax import lax
from jax.experimental import pallas as pl
from jax.experimental.pallas import tpu as pltpu
from jax.experimental.pallas import tpu_sc as plsc

N = 10000
DEG = 32
D = 128
NC = 2
NS = 16
NW = NC * NS
L = 16

CB = 4
IDX = CB * DEG
NCHUNK = N // CB
CPW = NCHUNK // NW
REM = NCHUNK - CPW * NW
MAXC = CPW + 1
NB = 3

_mesh = plsc.VectorSubcoreMesh(core_axis_name="c", subcore_axis_name="s")


@functools.partial(
    pl.kernel,
    mesh=_mesh,
    out_type=jax.ShapeDtypeStruct((N, D), jnp.float32),
    scratch_types=[
        pltpu.VMEM((MAXC * IDX,), jnp.int32),
        pltpu.VMEM((NB, IDX, D), jnp.float32),
        pltpu.VMEM((NB, CB, D), jnp.float32),
        pltpu.SemaphoreType.DMA,
        pltpu.SemaphoreType.DMA,
        pltpu.SemaphoreType.DMA,
        pltpu.SemaphoreType.DMA,
        pltpu.SemaphoreType.DMA,
        pltpu.SemaphoreType.DMA,
    ],
)
def _aggregate(idx_hbm, table_hbm, out_hbm, idx_v, rows_v, acc_v,
               gsem0, gsem1, gsem2, osem0, osem1, osem2):
    c32 = jnp.int32
    wid = lax.axis_index("s") * c32(NC) + lax.axis_index("c")
    start = wid * c32(CPW) + jnp.minimum(wid, c32(REM))
    cnt = c32(CPW) + jnp.where(wid < c32(REM), c32(1), c32(0))
    base = jnp.minimum(start, c32(NCHUNK - MAXC))
    off = start - base
    pltpu.sync_copy(idx_hbm.at[pl.ds(base * c32(IDX), MAXC * IDX)], idx_v)

    gsems = (gsem0, gsem1, gsem2)
    osems = (osem0, osem1, osem2)

    def gather(i, b):
        return pltpu.make_async_copy(
            table_hbm.at[idx_v.at[pl.ds((off + i) * c32(IDX), IDX)]],
            rows_v.at[c32(b)], gsems[b])

    def outcopy(i, b):
        return pltpu.make_async_copy(
            acc_v.at[c32(b)], out_hbm.at[pl.ds((start + i) * c32(CB), CB)],
            osems[b])

    for b in range(NB):
        gather(c32(b), b).start()

    def ring(j, _):
        for b in range(NB):
            i = j * c32(NB) + c32(b)

            @pl.when(i < cnt)
            def _():
                gather(i, b).wait()

                @pl.when(i >= c32(NB))
                def _():
                    outcopy(i - c32(NB), b).wait()

                def node_body(n, carry):
                    r0 = n * c32(DEG)
                    for half in range(2):
                        vs = range(half * 4, half * 4 + 4)
                        accs = {v: rows_v[b, r0, pl.ds(v * L, L)]
                                for v in vs}
                        for g in range(1, DEG):
                            for v in vs:
                                accs[v] = accs[v] + rows_v[b, r0 + c32(g),
                                                           pl.ds(v * L, L)]
                        for v in vs:
                            acc_v[b, n, pl.ds(v * L, L)] = (
                                accs[v] * (1.0 / DEG))
                    return carry

                lax.fori_loop(c32(0), c32(CB), node_body, c32(0))
                outcopy(i, b).start()

                @pl.when(i + c32(NB) < cnt)
                def _():
                    gather(i + c32(NB), b).start()
        return c32(0)

    lax.fori_loop(c32(0), c32((MAXC + NB - 1) // NB), ring, c32(0))
    for b in range(NB):
        outcopy(c32(0), b).wait()


def kernel(nodes, to_neighs, u2e_weight):
    del nodes
    idx = to_neighs.reshape(-1).astype(jnp.int32)
    table = u2e_weight.astype(jnp.float32)
    return _aggregate(idx, table)


# speedup vs baseline: 1.3514x; 1.0167x over previous
import functools

import jax
import jax.numpy as jnp
from jax import lax
from jax.experimental import pallas as pl
from jax.experimental.pallas import tpu as pltpu
from jax.experimental.pallas import tpu_sc as plsc

N = 10000
DEG = 32
D = 128
NC = 2
NS = 16
NW = NC * NS
L = 16

CB = 4
IDX = CB * DEG
NCHUNK = N // CB
CPW = NCHUNK // NW
REM = NCHUNK - CPW * NW
MAXC = CPW + 1
NB = 3

_mesh = plsc.VectorSubcoreMesh(core_axis_name="c", subcore_axis_name="s")


@functools.partial(
    pl.kernel,
    mesh=_mesh,
    out_type=jax.ShapeDtypeStruct((N, D), jnp.float32),
    scratch_types=[
        pltpu.VMEM((MAXC * IDX,), jnp.int32),
        pltpu.VMEM((NB, IDX, D), jnp.float32),
        pltpu.VMEM((NB, CB, D), jnp.float32),
        pltpu.SemaphoreType.DMA,
        pltpu.SemaphoreType.DMA,
        pltpu.SemaphoreType.DMA,
        pltpu.SemaphoreType.DMA,
        pltpu.SemaphoreType.DMA,
        pltpu.SemaphoreType.DMA,
        pltpu.SemaphoreType.DMA,
        pltpu.SemaphoreType.DMA,
        pltpu.SemaphoreType.DMA,
    ],
)
def _aggregate(idx_hbm, table_hbm, out_hbm, idx_v, rows_v, acc_v,
               gsem0, gsem1, gsem2, gsem3, gsem4, gsem5, osem0, osem1, osem2):
    c32 = jnp.int32
    wid = lax.axis_index("s") * c32(NC) + lax.axis_index("c")
    start = wid * c32(CPW) + jnp.minimum(wid, c32(REM))
    cnt = c32(CPW) + jnp.where(wid < c32(REM), c32(1), c32(0))
    base = jnp.minimum(start, c32(NCHUNK - MAXC))
    off = start - base
    pltpu.sync_copy(idx_hbm.at[pl.ds(base * c32(IDX), MAXC * IDX)], idx_v)

    gsems = ((gsem0, gsem3), (gsem1, gsem4), (gsem2, gsem5))
    osems = (osem0, osem1, osem2)

    def gather_half(i, b, h):
        return pltpu.make_async_copy(
            table_hbm.at[idx_v.at[pl.ds((off + i) * c32(IDX) + c32(h * 64),
                                        64)]],
            rows_v.at[c32(b), pl.ds(h * 64, 64)], gsems[b][h])

    class _Gather:
        def __init__(self, i, b):
            self.descs = [gather_half(i, b, h) for h in range(2)]

        def start(self):
            for d in self.descs:
                d.start()

        def wait(self):
            for d in self.descs:
                d.wait()

    gather = _Gather

    def outcopy(i, b):
        return pltpu.make_async_copy(
            acc_v.at[c32(b)], out_hbm.at[pl.ds((start + i) * c32(CB), CB)],
            osems[b])

    for b in range(NB):
        gather(c32(b), b).start()

    def ring(j, _):
        for b in range(NB):
            i = j * c32(NB) + c32(b)

            @pl.when(i < cnt)
            def _():
                gather(i, b).wait()

                @pl.when(i >= c32(NB))
                def _():
                    outcopy(i - c32(NB), b).wait()

                def node_body(n, carry):
                    r0 = n * c32(DEG)
                    for half in range(2):
                        vs = range(half * 4, half * 4 + 4)
                        accs = {v: rows_v[b, r0, pl.ds(v * L, L)]
                                for v in vs}
                        for g in range(1, DEG):
                            for v in vs:
                                accs[v] = accs[v] + rows_v[b, r0 + c32(g),
                                                           pl.ds(v * L, L)]
                        for v in vs:
                            acc_v[b, n, pl.ds(v * L, L)] = (
                                accs[v] * (1.0 / DEG))
                    return carry

                lax.fori_loop(c32(0), c32(CB), node_body, c32(0))
                outcopy(i, b).start()

                @pl.when(i + c32(NB) < cnt)
                def _():
                    gather(i + c32(NB), b).start()
        return c32(0)

    lax.fori_loop(c32(0), c32((MAXC + NB - 1) // NB), ring, c32(0))
    for b in range(NB):
        outcopy(c32(0), b).wait()


def kernel(nodes, to_neighs, u2e_weight):
    del nodes
    idx = to_neighs.reshape(-1).astype(jnp.int32)
    table = u2e_weight.astype(jnp.float32)
    return _aggregate(idx, table)
